# baseline (speedup 1.0000x reference)
import functools

import jax
import jax.numpy as jnp
from jax import lax
from jax.experimental import pallas as pl
from jax.experimental.pallas import tpu as pltpu
from jax.experimental.pallas import tpu_sc as plsc


def _gather_rows_sc(table, idx):
    num_rows = idx.shape[0]
    d = table.shape[1]
    mesh = plsc.VectorSubcoreMesh(core_axis_name="c", subcore_axis_name="s")
    num_workers = mesh.num_cores * mesh.num_subcores
    per_worker = num_rows // num_workers

    @functools.partial(
        pl.kernel,
        out_type=jax.ShapeDtypeStruct((num_rows, d), table.dtype),
        mesh=mesh,
        scratch_types=[
            pltpu.VMEM((per_worker,), jnp.int32),
            pltpu.VMEM((per_worker, d), table.dtype),
            pltpu.SemaphoreType.DMA,
        ],
    )
    def gather_kernel(table_hbm, idx_hbm, out_hbm, idx_v, rows_v, sem):
        wid = lax.axis_index("s") * mesh.num_cores + lax.axis_index("c")
        base = wid * per_worker
        pltpu.sync_copy(idx_hbm.at[pl.ds(base, per_worker)], idx_v)
        pltpu.async_copy(table_hbm.at[idx_v], rows_v, sem).wait()
        pltpu.sync_copy(rows_v, out_hbm.at[pl.ds(base, per_worker)])

    return gather_kernel(table, idx)


def _assemble_body(idx_ref, text_ref, tbl_ref, gini_ref, wt_ref, bt_ref,
                   wg_ref, bg_ref, text_out_ref, mem_out_ref,
                   t_v, g_v, copy_sem, tbl_sem, patch_sem):
    b, l, d = text_ref.shape
    m = tbl_ref.shape[0]

    n_tbl_chunks = 10
    rows_per_chunk = m // n_tbl_chunks
    tail = m - n_tbl_chunks * rows_per_chunk
    tbl_copies = []
    for k in range(n_tbl_chunks):
        c = pltpu.make_async_copy(
            tbl_ref.at[pl.ds(k * rows_per_chunk, rows_per_chunk)],
            mem_out_ref.at[pl.ds(k * rows_per_chunk, rows_per_chunk)],
            tbl_sem)
        c.start()
        tbl_copies.append(c)
    if tail:
        c = pltpu.make_async_copy(
            tbl_ref.at[pl.ds(n_tbl_chunks * rows_per_chunk, tail)],
            mem_out_ref.at[pl.ds(n_tbl_chunks * rows_per_chunk, tail)],
            tbl_sem)
        c.start()
        tbl_copies.append(c)

    n_text_chunks = 16
    bpc = b // n_text_chunks
    text_copies = []
    for k in range(n_text_chunks):
        c = pltpu.make_async_copy(
            text_ref.at[pl.ds(k * bpc, bpc)],
            text_out_ref.at[pl.ds(k * bpc, bpc)],
            copy_sem)
        c.start()
        text_copies.append(c)

    g = gini_ref[...]
    dims = (((1,), (1,)), ((), ()))
    t_v[:, 0, :] = (lax.dot_general(g, wt_ref[...], dims,
                                    preferred_element_type=jnp.float32)
                    + bt_ref[...])
    g_v[:, 0, :] = (lax.dot_general(g, wg_ref[...], dims,
                                    preferred_element_type=jnp.float32)
                    + bg_ref[...])

    for c in text_copies:
        c.wait()
    t_copy = pltpu.make_async_copy(t_v, text_out_ref.at[:, pl.ds(0, 1), :],
                                   copy_sem)
    t_copy.start()

    for c in tbl_copies:
        c.wait()

    def patch_start(j, carry):
        r = idx_ref[j]
        pltpu.make_async_copy(g_v.at[pl.ds(j, 1)],
                              mem_out_ref.at[pl.ds(r, 1)], patch_sem).start()
        return carry

    lax.fori_loop(0, b, patch_start, 0)

    def patch_wait(j, carry):
        r = idx_ref[j]
        pltpu.make_async_copy(g_v.at[pl.ds(j, 1)],
                              mem_out_ref.at[pl.ds(r, 1)], patch_sem).wait()
        return carry

    lax.fori_loop(0, b, patch_wait, 0)

    t_copy.wait()


def _assemble(text, table, graph_ini, w_text, b_text, w_graph, b_graph, idx):
    b, l, d = text.shape
    m = table.shape[0]
    f32 = jnp.float32
    return pl.pallas_call(
        _assemble_body,
        out_shape=(jax.ShapeDtypeStruct((b, l, d), f32),
                   jax.ShapeDtypeStruct((m, 1, d), f32)),
        in_specs=[
            pl.BlockSpec(memory_space=pltpu.SMEM),
            pl.BlockSpec(memory_space=pl.ANY),
            pl.BlockSpec(memory_space=pl.ANY),
            pl.BlockSpec(memory_space=pltpu.VMEM),
            pl.BlockSpec(memory_space=pltpu.VMEM),
            pl.BlockSpec(memory_space=pltpu.VMEM),
            pl.BlockSpec(memory_space=pltpu.VMEM),
            pl.BlockSpec(memory_space=pltpu.VMEM),
        ],
        out_specs=(pl.BlockSpec(memory_space=pl.ANY),
                   pl.BlockSpec(memory_space=pl.ANY)),
        scratch_shapes=[
            pltpu.VMEM((b, 1, d), f32),
            pltpu.VMEM((b, 1, d), f32),
            pltpu.SemaphoreType.DMA,
            pltpu.SemaphoreType.DMA,
            pltpu.SemaphoreType.DMA,
        ],
    )(idx, text, table.reshape(m, 1, d), graph_ini,
      w_text, b_text, w_graph, b_graph)


def kernel(text, all_user_feature, user_neighbor_index,
           W_text, b_text, W_graph, b_graph):
    idx = user_neighbor_index[:, 0].astype(jnp.int32)
    graph_ini = _gather_rows_sc(all_user_feature, idx)
    text_out, new_mem3 = _assemble(
        text, all_user_feature, graph_ini,
        W_text, b_text.reshape(1, -1), W_graph, b_graph.reshape(1, -1), idx)
    return text_out, new_mem3.reshape(all_user_feature.shape)


# speedup vs baseline: 1.0581x; 1.0581x over previous
import functools

import jax
import jax.numpy as jnp
from jax import lax
from jax.experimental import pallas as pl
from jax.experimental.pallas import tpu as pltpu
from jax.experimental.pallas import tpu_sc as plsc


def _gather_rows_sc(table, idx):
    num_rows = idx.shape[0]
    d = table.shape[1]
    mesh = plsc.VectorSubcoreMesh(core_axis_name="c", subcore_axis_name="s")
    num_workers = mesh.num_cores * mesh.num_subcores
    per_worker = num_rows // num_workers

    @functools.partial(
        pl.kernel,
        out_type=jax.ShapeDtypeStruct((num_rows, d), table.dtype),
        mesh=mesh,
        scratch_types=[
            pltpu.VMEM((per_worker,), jnp.int32),
            pltpu.VMEM((per_worker, d), table.dtype),
            pltpu.SemaphoreType.DMA,
        ],
    )
    def gather_kernel(table_hbm, idx_hbm, out_hbm, idx_v, rows_v, sem):
        wid = lax.axis_index("s") * mesh.num_cores + lax.axis_index("c")
        base = wid * per_worker
        pltpu.sync_copy(idx_hbm.at[pl.ds(base, per_worker)], idx_v)
        pltpu.async_copy(table_hbm.at[idx_v], rows_v, sem).wait()
        pltpu.sync_copy(rows_v, out_hbm.at[pl.ds(base, per_worker)])

    return gather_kernel(table, idx)


_N_TBL_CHUNKS = 10
_N_TEXT_CHUNKS = 16


def _assemble_body(order_ref, ssub_ref, slot_ref, ublk_ref, nu_ref,
                   text_ref, tbl_ref, gini_ref, wt_ref, bt_ref,
                   wg_ref, bg_ref, text_out_ref, mem_out_ref,
                   t_v, g_v, blocks_v, text_sem, tbl_sem, blk_sem,
                   slot0_sem):
    b, l, d = text_ref.shape
    mb = tbl_ref.shape[0]
    n_u = nu_ref[0]

    rows_per_chunk = mb // _N_TBL_CHUNKS
    tbl_copies = []
    for k in range(_N_TBL_CHUNKS):
        c = pltpu.make_async_copy(
            tbl_ref.at[pl.ds(k * rows_per_chunk, rows_per_chunk)],
            mem_out_ref.at[pl.ds(k * rows_per_chunk, rows_per_chunk)],
            tbl_sem)
        c.start()
        tbl_copies.append(c)

    bpc = b // _N_TEXT_CHUNKS
    text_copies = []
    for k in range(_N_TEXT_CHUNKS):
        c = pltpu.make_async_copy(
            text_ref.at[pl.ds(k * bpc, bpc)],
            text_out_ref.at[pl.ds(k * bpc, bpc)],
            text_sem)
        c.start()
        text_copies.append(c)

    def blk_read_start(k, carry):
        pltpu.make_async_copy(tbl_ref.at[pl.ds(ublk_ref[k], 1)],
                              blocks_v.at[pl.ds(k, 1)], blk_sem).start()
        return carry

    lax.fori_loop(0, n_u, blk_read_start, 0)

    g = gini_ref[...]
    dims = (((1,), (1,)), ((), ()))
    t_v[:, 0, :] = (lax.dot_general(g, wt_ref[...], dims,
                                    preferred_element_type=jnp.float32)
                    + bt_ref[...])
    g_v[:, 0, :] = (lax.dot_general(g, wg_ref[...], dims,
                                    preferred_element_type=jnp.float32)
                    + bg_ref[...])

    def blk_read_wait(k, carry):
        pltpu.make_async_copy(tbl_ref.at[pl.ds(ublk_ref[k], 1)],
                              blocks_v.at[pl.ds(k, 1)], blk_sem).wait()
        return carry

    lax.fori_loop(0, n_u, blk_read_wait, 0)

    sub_iota = lax.broadcasted_iota(jnp.int32, (1, 8, 1), 1)

    def patch(j, carry):
        oj = order_ref[j]
        sj = slot_ref[j]
        row = g_v[pl.ds(oj, 1), 0, :]
        cur = blocks_v[pl.ds(sj, 1), :, :]
        mask = sub_iota == ssub_ref[j]
        blocks_v[pl.ds(sj, 1), :, :] = jnp.where(mask, row[:, None, :], cur)
        return carry

    lax.fori_loop(0, b, patch, 0)

    for c in text_copies:
        c.wait()
    t_copy = pltpu.make_async_copy(t_v, text_out_ref.at[:, pl.ds(0, 1), :],
                                   slot0_sem)
    t_copy.start()

    for c in tbl_copies:
        c.wait()

    def blk_write_start(k, carry):
        pltpu.make_async_copy(blocks_v.at[pl.ds(k, 1)],
                              mem_out_ref.at[pl.ds(ublk_ref[k], 1)],
                              blk_sem).start()
        return carry

    lax.fori_loop(0, n_u, blk_write_start, 0)

    def blk_write_wait(k, carry):
        pltpu.make_async_copy(blocks_v.at[pl.ds(k, 1)],
                              mem_out_ref.at[pl.ds(ublk_ref[k], 1)],
                              blk_sem).wait()
        return carry

    lax.fori_loop(0, n_u, blk_write_wait, 0)

    t_copy.wait()


def _assemble(text, tbl3, graph_ini, w_text, b_text, w_graph, b_graph,
              order, ssub, slot, ublk, n_u):
    b, l, d = text.shape
    mb = tbl3.shape[0]
    f32 = jnp.float32
    return pl.pallas_call(
        _assemble_body,
        out_shape=(jax.ShapeDtypeStruct((b, l, d), f32),
                   jax.ShapeDtypeStruct((mb, 8, d), f32)),
        in_specs=[
            pl.BlockSpec(memory_space=pltpu.SMEM),
            pl.BlockSpec(memory_space=pltpu.SMEM),
            pl.BlockSpec(memory_space=pltpu.SMEM),
            pl.BlockSpec(memory_space=pltpu.SMEM),
            pl.BlockSpec(memory_space=pltpu.SMEM),
            pl.BlockSpec(memory_space=pl.ANY),
            pl.BlockSpec(memory_space=pl.ANY),
            pl.BlockSpec(memory_space=pltpu.VMEM),
            pl.BlockSpec(memory_space=pltpu.VMEM),
            pl.BlockSpec(memory_space=pltpu.VMEM),
            pl.BlockSpec(memory_space=pltpu.VMEM),
            pl.BlockSpec(memory_space=pltpu.VMEM),
        ],
        out_specs=(pl.BlockSpec(memory_space=pl.ANY),
                   pl.BlockSpec(memory_space=pl.ANY)),
        scratch_shapes=[
            pltpu.VMEM((b, 1, d), f32),
            pltpu.VMEM((b, 1, d), f32),
            pltpu.VMEM((b, 8, d), f32),
            pltpu.SemaphoreType.DMA,
            pltpu.SemaphoreType.DMA,
            pltpu.SemaphoreType.DMA,
            pltpu.SemaphoreType.DMA,
        ],
    )(order, ssub, slot, ublk, n_u, text, tbl3, graph_ini,
      w_text, b_text, w_graph, b_graph)


def kernel(text, all_user_feature, user_neighbor_index,
           W_text, b_text, W_graph, b_graph):
    m, d = all_user_feature.shape
    b = text.shape[0]
    idx = user_neighbor_index[:, 0].astype(jnp.int32)
    graph_ini = _gather_rows_sc(all_user_feature, idx)

    blk = idx // 8
    sub = idx % 8
    order = jnp.argsort(blk).astype(jnp.int32)
    sblk = blk[order]
    ssub = sub[order]
    leader = jnp.concatenate(
        [jnp.ones((1,), jnp.int32),
         (sblk[1:] != sblk[:-1]).astype(jnp.int32)])
    slot = jnp.cumsum(leader, dtype=jnp.int32) - 1
    n_u = slot[-1:] + 1
    ublk = jnp.zeros((b,), jnp.int32).at[slot].set(sblk)

    tbl3 = all_user_feature.reshape(m // 8, 8, d)
    text_out, new_mem3 = _assemble(
        text, tbl3, graph_ini,
        W_text, b_text.reshape(1, -1), W_graph, b_graph.reshape(1, -1),
        order, ssub, slot, ublk, n_u)
    return text_out, new_mem3.reshape(m, d)


# speedup vs baseline: 20.8712x; 19.7254x over previous
import functools

import jax
import jax.numpy as jnp
from jax import lax
from jax.experimental import pallas as pl
from jax.experimental.pallas import tpu as pltpu
from jax.experimental.pallas import tpu_sc as plsc


def _gather_rows_sc(table, idx):
    num_rows = idx.shape[0]
    d = table.shape[1]
    mesh = plsc.VectorSubcoreMesh(core_axis_name="c", subcore_axis_name="s")
    num_workers = mesh.num_cores * mesh.num_subcores
    per_worker = num_rows // num_workers

    @functools.partial(
        pl.kernel,
        out_type=jax.ShapeDtypeStruct((num_rows, d), table.dtype),
        mesh=mesh,
        scratch_types=[
            pltpu.VMEM((per_worker,), jnp.int32),
            pltpu.VMEM((per_worker, d), table.dtype),
            pltpu.SemaphoreType.DMA,
        ],
    )
    def gather_kernel(table_hbm, idx_hbm, out_hbm, idx_v, rows_v, sem):
        wid = lax.axis_index("s") * mesh.num_cores + lax.axis_index("c")
        base = wid * per_worker
        pltpu.sync_copy(idx_hbm.at[pl.ds(base, per_worker)], idx_v)
        pltpu.async_copy(table_hbm.at[idx_v], rows_v, sem).wait()
        pltpu.sync_copy(rows_v, out_hbm.at[pl.ds(base, per_worker)])

    return gather_kernel(table, idx)


_TEXT_BB = 8
_TBL_BB = 125


def _text_body(gini_ref, wt_ref, bt_ref, text_ref, text_out_ref, t_all):
    i = pl.program_id(0)

    @pl.when(i == 0)
    def _():
        dims = (((1,), (1,)), ((), ()))
        t = (lax.dot_general(gini_ref[...], wt_ref[...], dims,
                             preferred_element_type=jnp.float32)
             + bt_ref[...])
        t_all[...] = t.reshape(t_all.shape)

    text_out_ref[...] = text_ref[...]
    text_out_ref[:, 0, :] = t_all[i]


def _text_kernel(text, gini, w_text, b_text):
    b, l, d = text.shape
    grid = b // _TEXT_BB
    return pl.pallas_call(
        _text_body,
        grid=(grid,),
        in_specs=[
            pl.BlockSpec((b, d), lambda i: (0, 0)),
            pl.BlockSpec((d, d), lambda i: (0, 0)),
            pl.BlockSpec((1, d), lambda i: (0, 0)),
            pl.BlockSpec((_TEXT_BB, l, d), lambda i: (i, 0, 0)),
        ],
        out_specs=pl.BlockSpec((_TEXT_BB, l, d), lambda i: (i, 0, 0)),
        out_shape=jax.ShapeDtypeStruct((b, l, d), jnp.float32),
        scratch_shapes=[pltpu.VMEM((grid, _TEXT_BB, d), jnp.float32)],
    )(gini, w_text, b_text, text)


def _table_body(order_ref, sblk_ref, ssub_ref, cs_ref, ce_ref,
                gini_ref, wg_ref, bg_ref, tbl_ref, mem_out_ref, g_v):
    i = pl.program_id(0)

    @pl.when(i == 0)
    def _():
        dims = (((1,), (1,)), ((), ()))
        g = (lax.dot_general(gini_ref[...], wg_ref[...], dims,
                             preferred_element_type=jnp.float32)
             + bg_ref[...])
        g_v[:, 0, :] = g

    mem_out_ref[...] = tbl_ref[...]

    sub_iota = lax.broadcasted_iota(jnp.int32, (1, 8, 1), 1)
    base = i * _TBL_BB

    def patch(j, carry):
        oj = order_ref[j]
        rl = sblk_ref[j] - base
        row = g_v[pl.ds(oj, 1), 0, :]
        cur = mem_out_ref[pl.ds(rl, 1), :, :]
        mask = sub_iota == ssub_ref[j]
        mem_out_ref[pl.ds(rl, 1), :, :] = jnp.where(
            mask, row[:, None, :], cur)
        return carry

    lax.fori_loop(cs_ref[i], ce_ref[i], patch, 0)


def _table_kernel(tbl3, gini, w_graph, b_graph, order, sblk, ssub, cs, ce):
    mb, _, d = tbl3.shape
    b = gini.shape[0]
    grid = mb // _TBL_BB
    return pl.pallas_call(
        _table_body,
        grid=(grid,),
        in_specs=[
            pl.BlockSpec(memory_space=pltpu.SMEM),
            pl.BlockSpec(memory_space=pltpu.SMEM),
            pl.BlockSpec(memory_space=pltpu.SMEM),
            pl.BlockSpec(memory_space=pltpu.SMEM),
            pl.BlockSpec(memory_space=pltpu.SMEM),
            pl.BlockSpec((b, d), lambda i: (0, 0)),
            pl.BlockSpec((d, d), lambda i: (0, 0)),
            pl.BlockSpec((1, d), lambda i: (0, 0)),
            pl.BlockSpec((_TBL_BB, 8, d), lambda i: (i, 0, 0)),
        ],
        out_specs=pl.BlockSpec((_TBL_BB, 8, d), lambda i: (i, 0, 0)),
        out_shape=jax.ShapeDtypeStruct((mb, 8, d), jnp.float32),
        scratch_shapes=[pltpu.VMEM((b, 1, d), jnp.float32)],
    )(order, sblk, ssub, cs, ce, gini, w_graph, b_graph, tbl3)


def kernel(text, all_user_feature, user_neighbor_index,
           W_text, b_text, W_graph, b_graph):
    m, d = all_user_feature.shape
    idx = user_neighbor_index[:, 0].astype(jnp.int32)
    graph_ini = _gather_rows_sc(all_user_feature, idx)

    blk = idx // 8
    sub = idx % 8
    order = jnp.argsort(blk).astype(jnp.int32)
    sblk = blk[order]
    ssub = sub[order]
    n_chunks = (m // 8) // _TBL_BB
    bounds = jnp.arange(n_chunks + 1, dtype=jnp.int32) * _TBL_BB
    edges = jnp.searchsorted(sblk, bounds).astype(jnp.int32)
    cs, ce = edges[:-1], edges[1:]

    tbl3 = all_user_feature.reshape(m // 8, 8, d)
    text_out = _text_kernel(text, graph_ini, W_text, b_text.reshape(1, -1))
    new_mem3 = _table_kernel(tbl3, graph_ini, W_graph,
                             b_graph.reshape(1, -1), order, sblk, ssub,
                             cs, ce)
    return text_out, new_mem3.reshape(m, d)


# speedup vs baseline: 21.1374x; 1.0128x over previous
import functools

import jax
import jax.numpy as jnp
from jax import lax
from jax.experimental import pallas as pl
from jax.experimental.pallas import tpu as pltpu
from jax.experimental.pallas import tpu_sc as plsc


def _gather_rows_sc(table, idx):
    num_rows = idx.shape[0]
    d = table.shape[1]
    mesh = plsc.VectorSubcoreMesh(core_axis_name="c", subcore_axis_name="s")
    num_workers = mesh.num_cores * mesh.num_subcores
    per_worker = num_rows // num_workers

    @functools.partial(
        pl.kernel,
        out_type=jax.ShapeDtypeStruct((num_rows, d), table.dtype),
        mesh=mesh,
        scratch_types=[
            pltpu.VMEM((per_worker,), jnp.int32),
            pltpu.VMEM((per_worker, d), table.dtype),
            pltpu.SemaphoreType.DMA,
        ],
    )
    def gather_kernel(table_hbm, idx_hbm, out_hbm, idx_v, rows_v, sem):
        wid = lax.axis_index("s") * mesh.num_cores + lax.axis_index("c")
        base = wid * per_worker
        pltpu.sync_copy(idx_hbm.at[pl.ds(base, per_worker)], idx_v)
        pltpu.async_copy(table_hbm.at[idx_v], rows_v, sem).wait()
        pltpu.sync_copy(rows_v, out_hbm.at[pl.ds(base, per_worker)])

    return gather_kernel(table, idx)


_TEXT_BB = 16
_TBL_BB = 250


def _text_body(gini_ref, wt_ref, bt_ref, text_ref, text_out_ref, t_all,
               sem):
    i = pl.program_id(0)

    @pl.when(i == 0)
    def _():
        dims = (((1,), (1,)), ((), ()))
        t = (lax.dot_general(gini_ref[...], wt_ref[...], dims,
                             preferred_element_type=jnp.float32)
             + bt_ref[...])
        t_all[...] = t.reshape(t_all.shape)

    c = pltpu.make_async_copy(text_ref, text_out_ref, sem)
    c.start()
    c.wait()
    text_out_ref[:, 0, :] = t_all[i]


def _text_kernel(text, gini, w_text, b_text):
    b, l, d = text.shape
    grid = b // _TEXT_BB
    return pl.pallas_call(
        _text_body,
        grid=(grid,),
        in_specs=[
            pl.BlockSpec((b, d), lambda i: (0, 0)),
            pl.BlockSpec((d, d), lambda i: (0, 0)),
            pl.BlockSpec((1, d), lambda i: (0, 0)),
            pl.BlockSpec((_TEXT_BB, l, d), lambda i: (i, 0, 0)),
        ],
        out_specs=pl.BlockSpec((_TEXT_BB, l, d), lambda i: (i, 0, 0)),
        out_shape=jax.ShapeDtypeStruct((b, l, d), jnp.float32),
        scratch_shapes=[pltpu.VMEM((grid, _TEXT_BB, d), jnp.float32),
                        pltpu.SemaphoreType.DMA],
    )(gini, w_text, b_text, text)


def _table_body(order_ref, sblk_ref, ssub_ref, cs_ref, ce_ref,
                gini_ref, wg_ref, bg_ref, tbl_ref, mem_out_ref, g_v, sem):
    i = pl.program_id(0)

    @pl.when(i == 0)
    def _():
        dims = (((1,), (1,)), ((), ()))
        g = (lax.dot_general(gini_ref[...], wg_ref[...], dims,
                             preferred_element_type=jnp.float32)
             + bg_ref[...])
        g_v[:, 0, :] = g

    c = pltpu.make_async_copy(tbl_ref, mem_out_ref, sem)
    c.start()
    c.wait()

    sub_iota = lax.broadcasted_iota(jnp.int32, (1, 8, 1), 1)
    base = i * _TBL_BB

    def patch(j, carry):
        oj = order_ref[j]
        rl = sblk_ref[j] - base
        row = g_v[pl.ds(oj, 1), 0, :]
        cur = mem_out_ref[pl.ds(rl, 1), :, :]
        mask = sub_iota == ssub_ref[j]
        mem_out_ref[pl.ds(rl, 1), :, :] = jnp.where(
            mask, row[:, None, :], cur)
        return carry

    lax.fori_loop(cs_ref[i], ce_ref[i], patch, 0)


def _table_kernel(tbl3, gini, w_graph, b_graph, order, sblk, ssub, cs, ce):
    mb, _, d = tbl3.shape
    b = gini.shape[0]
    grid = mb // _TBL_BB
    return pl.pallas_call(
        _table_body,
        grid=(grid,),
        in_specs=[
            pl.BlockSpec(memory_space=pltpu.SMEM),
            pl.BlockSpec(memory_space=pltpu.SMEM),
            pl.BlockSpec(memory_space=pltpu.SMEM),
            pl.BlockSpec(memory_space=pltpu.SMEM),
            pl.BlockSpec(memory_space=pltpu.SMEM),
            pl.BlockSpec((b, d), lambda i: (0, 0)),
            pl.BlockSpec((d, d), lambda i: (0, 0)),
            pl.BlockSpec((1, d), lambda i: (0, 0)),
            pl.BlockSpec((_TBL_BB, 8, d), lambda i: (i, 0, 0)),
        ],
        out_specs=pl.BlockSpec((_TBL_BB, 8, d), lambda i: (i, 0, 0)),
        out_shape=jax.ShapeDtypeStruct((mb, 8, d), jnp.float32),
        scratch_shapes=[pltpu.VMEM((b, 1, d), jnp.float32),
                        pltpu.SemaphoreType.DMA],
    )(order, sblk, ssub, cs, ce, gini, w_graph, b_graph, tbl3)


def kernel(text, all_user_feature, user_neighbor_index,
           W_text, b_text, W_graph, b_graph):
    m, d = all_user_feature.shape
    idx = user_neighbor_index[:, 0].astype(jnp.int32)
    graph_ini = _gather_rows_sc(all_user_feature, idx)

    blk = idx // 8
    sub = idx % 8
    order = jnp.argsort(blk).astype(jnp.int32)
    sblk = blk[order]
    ssub = sub[order]
    n_chunks = (m // 8) // _TBL_BB
    bounds = jnp.arange(n_chunks + 1, dtype=jnp.int32) * _TBL_BB
    edges = jnp.searchsorted(sblk, bounds).astype(jnp.int32)
    cs, ce = edges[:-1], edges[1:]

    tbl3 = all_user_feature.reshape(m // 8, 8, d)
    text_out = _text_kernel(text, graph_ini, W_text, b_text.reshape(1, -1))
    new_mem3 = _table_kernel(tbl3, graph_ini, W_graph,
                             b_graph.reshape(1, -1), order, sblk, ssub,
                             cs, ce)
    return text_out, new_mem3.reshape(m, d)


# speedup vs baseline: 21.2434x; 1.0050x over previous
import functools

import jax
import jax.numpy as jnp
from jax import lax
from jax.experimental import pallas as pl
from jax.experimental.pallas import tpu as pltpu
from jax.experimental.pallas import tpu_sc as plsc


def _gather_rows_sc(table, idx):
    num_rows = idx.shape[0]
    d = table.shape[1]
    mesh = plsc.VectorSubcoreMesh(core_axis_name="c", subcore_axis_name="s")
    num_workers = mesh.num_cores * mesh.num_subcores
    per_worker = num_rows // num_workers

    @functools.partial(
        pl.kernel,
        out_type=jax.ShapeDtypeStruct((num_rows, d), table.dtype),
        mesh=mesh,
        scratch_types=[
            pltpu.VMEM((per_worker,), jnp.int32),
            pltpu.VMEM((per_worker, d), table.dtype),
            pltpu.SemaphoreType.DMA,
        ],
    )
    def gather_kernel(table_hbm, idx_hbm, out_hbm, idx_v, rows_v, sem):
        wid = lax.axis_index("s") * mesh.num_cores + lax.axis_index("c")
        base = wid * per_worker
        pltpu.sync_copy(idx_hbm.at[pl.ds(base, per_worker)], idx_v)
        pltpu.async_copy(table_hbm.at[idx_v], rows_v, sem).wait()
        pltpu.sync_copy(rows_v, out_hbm.at[pl.ds(base, per_worker)])

    return gather_kernel(table, idx)


_TEXT_BB = 8
_TBL_BB = 125
_NBUF = 6
_DEPTH = 3


def _assemble_body(sblk_ref, ssub_ref, oblk_ref, osub_ref, cs_ref, ce_ref,
                   text_ref, tbl_ref, gini_ref, wt_ref, bt_ref,
                   wg_ref, bg_ref, text_out_ref, mem_out_ref,
                   tbuf, bbuf, t_all, g_all, tin_sem, tout_sem,
                   bin_sem, bout_sem):
    b, l, d = text_ref.shape
    mb = tbl_ref.shape[0]
    nt = b // _TEXT_BB
    nb = mb // _TBL_BB

    def t_in(k):
        return pltpu.make_async_copy(
            text_ref.at[pl.ds(k * _TEXT_BB, _TEXT_BB)],
            tbuf.at[k % _NBUF], tin_sem.at[k % _NBUF])

    def t_out(k):
        return pltpu.make_async_copy(
            tbuf.at[k % _NBUF],
            text_out_ref.at[pl.ds(k * _TEXT_BB, _TEXT_BB)],
            tout_sem.at[k % _NBUF])

    def b_in(k):
        return pltpu.make_async_copy(
            tbl_ref.at[pl.ds(k * _TBL_BB, _TBL_BB)],
            bbuf.at[k % _NBUF], bin_sem.at[k % _NBUF])

    def b_out(k):
        return pltpu.make_async_copy(
            bbuf.at[k % _NBUF],
            mem_out_ref.at[pl.ds(k * _TBL_BB, _TBL_BB)],
            bout_sem.at[k % _NBUF])

    sub_iota = lax.broadcasted_iota(jnp.int32, (1, 8, 1), 1)

    def patch_tbl(k):
        s = k % _NBUF
        base = k * _TBL_BB

        def patch(j, carry):
            ob = oblk_ref[j]
            rl = sblk_ref[j] - base
            g_blk = g_all[pl.ds(ob, 1), :, :]
            rmask = sub_iota == osub_ref[j]
            row = jnp.sum(jnp.where(rmask, g_blk, 0.0), axis=1)
            cur = bbuf[s, pl.ds(rl, 1), :, :]
            mask = sub_iota == ssub_ref[j]
            bbuf[s, pl.ds(rl, 1), :, :] = jnp.where(
                mask, row[:, None, :], cur)
            return carry

        lax.fori_loop(cs_ref[k], ce_ref[k], patch, 0)

    for k in range(_DEPTH):
        t_in(k).start()
        b_in(k).start()

    dims = (((1,), (1,)), ((), ()))
    g0 = gini_ref[...]
    t = (lax.dot_general(g0, wt_ref[...], dims,
                         preferred_element_type=jnp.float32) + bt_ref[...])
    t_all[...] = t.reshape(t_all.shape)
    g = (lax.dot_general(g0, wg_ref[...], dims,
                         preferred_element_type=jnp.float32) + bg_ref[...])
    g_all[...] = g.reshape(g_all.shape)

    n_iter = max(nt, nb) + _DEPTH
    for k in range(n_iter):
        if _DEPTH <= k < nt:
            if k >= _NBUF:
                t_out(k - _NBUF).wait()
            t_in(k).start()
        if _DEPTH <= k < nb:
            if k >= _NBUF:
                b_out(k - _NBUF).wait()
            b_in(k).start()
        j = k - _DEPTH
        if 0 <= j < nt:
            t_in(j).wait()
            tbuf[j % _NBUF, :, 0, :] = t_all[j]
            t_out(j).start()
        if 0 <= j < nb:
            b_in(j).wait()
            patch_tbl(j)
            b_out(j).start()

    for k in range(max(0, nt - _NBUF), nt):
        t_out(k).wait()
    for k in range(max(0, nb - _NBUF), nb):
        b_out(k).wait()


def _assemble(text, tbl3, graph_ini, w_text, b_text, w_graph, b_graph,
              sblk, ssub, oblk, osub, cs, ce):
    b, l, d = text.shape
    mb = tbl3.shape[0]
    f32 = jnp.float32
    return pl.pallas_call(
        _assemble_body,
        out_shape=(jax.ShapeDtypeStruct((b, l, d), f32),
                   jax.ShapeDtypeStruct((mb, 8, d), f32)),
        in_specs=[
            pl.BlockSpec(memory_space=pltpu.SMEM),
            pl.BlockSpec(memory_space=pltpu.SMEM),
            pl.BlockSpec(memory_space=pltpu.SMEM),
            pl.BlockSpec(memory_space=pltpu.SMEM),
            pl.BlockSpec(memory_space=pltpu.SMEM),
            pl.BlockSpec(memory_space=pltpu.SMEM),
            pl.BlockSpec(memory_space=pl.ANY),
            pl.BlockSpec(memory_space=pl.ANY),
            pl.BlockSpec(memory_space=pltpu.VMEM),
            pl.BlockSpec(memory_space=pltpu.VMEM),
            pl.BlockSpec(memory_space=pltpu.VMEM),
            pl.BlockSpec(memory_space=pltpu.VMEM),
            pl.BlockSpec(memory_space=pltpu.VMEM),
        ],
        out_specs=(pl.BlockSpec(memory_space=pl.ANY),
                   pl.BlockSpec(memory_space=pl.ANY)),
        scratch_shapes=[
            pltpu.VMEM((_NBUF, _TEXT_BB, l, d), f32),
            pltpu.VMEM((_NBUF, _TBL_BB, 8, d), f32),
            pltpu.VMEM((b // 8, 8, d), f32),
            pltpu.VMEM((b // 8, 8, d), f32),
            pltpu.SemaphoreType.DMA((_NBUF,)),
            pltpu.SemaphoreType.DMA((_NBUF,)),
            pltpu.SemaphoreType.DMA((_NBUF,)),
            pltpu.SemaphoreType.DMA((_NBUF,)),
        ],
    )(sblk, ssub, oblk, osub, cs, ce, text, tbl3, graph_ini,
      w_text, b_text, w_graph, b_graph)


def kernel(text, all_user_feature, user_neighbor_index,
           W_text, b_text, W_graph, b_graph):
    m, d = all_user_feature.shape
    idx = user_neighbor_index[:, 0].astype(jnp.int32)
    graph_ini = _gather_rows_sc(all_user_feature, idx)

    blk = idx // 8
    sub = idx % 8
    order = jnp.argsort(blk).astype(jnp.int32)
    sblk = blk[order]
    ssub = sub[order]
    oblk = order // 8
    osub = order % 8
    n_chunks = (m // 8) // _TBL_BB
    bounds = jnp.arange(n_chunks + 1, dtype=jnp.int32) * _TBL_BB
    edges = jnp.searchsorted(sblk, bounds).astype(jnp.int32)
    cs, ce = edges[:-1], edges[1:]

    tbl3 = all_user_feature.reshape(m // 8, 8, d)
    text_out, new_mem3 = _assemble(
        text, tbl3, graph_ini,
        W_text, b_text.reshape(1, -1), W_graph, b_graph.reshape(1, -1),
        sblk, ssub, oblk, osub, cs, ce)
    return text_out, new_mem3.reshape(m, d)
